# Initial kernel scaffold; baseline (speedup 1.0000x reference)
import jax
import jax.numpy as jnp
from jax.experimental import pallas as pl


def kernel(x, edge_index, W1, b1, W2, b2, W3, b3, W4, b4, Wl1, bl1):
    raise NotImplementedError("write your pallas kernel here")


# speedup vs baseline: 4.0988x; 4.0988x over previous
import functools

import jax
import jax.numpy as jnp
from jax import lax
from jax.experimental import pallas as pl
from jax.experimental.pallas import tpu as pltpu
from jax.experimental.pallas import tpu_sc as plsc

N = 10000
F = 128
E = 320000
NCLS = 2

CH = 128
NWORK = 32
CHUNKS = 79
EPW = CH * CHUNKS
EPAD = EPW * NWORK
NPAD = 10112
RPT = NPAD // 16

_THETAS = ((3.0, -3.0, 0.75), (0.0, 3.0, -1.5), (0.0, 0.0, 0.75))


def _mlp_body(x_ref, w1_ref, b1_ref, w2_ref, b2_ref, wl1_ref, bl1_ref,
              h_ref, ix_ref):
    xb = x_ref[...]
    h = jnp.maximum(
        jnp.dot(xb, w1_ref[...], preferred_element_type=jnp.float32)
        + b1_ref[...], 0.0)
    h = jnp.maximum(
        jnp.dot(h, w2_ref[...], preferred_element_type=jnp.float32)
        + b2_ref[...], 0.0)
    h_ref[...] = h
    ix_ref[...] = (jnp.dot(xb, wl1_ref[...], preferred_element_type=jnp.float32)
                   + bl1_ref[...])


_ROWS_B = 1000


def _mlp(x, w1t, b1, w2t, b2, wl1t, bl1):
    grid = (N // _ROWS_B,)
    full = lambda shape: pl.BlockSpec(shape, lambda i: (0, 0))
    return pl.pallas_call(
        _mlp_body,
        grid=grid,
        in_specs=[
            pl.BlockSpec((_ROWS_B, F), lambda i: (i, 0)),
            full((F, F)), full((1, F)),
            full((F, F)), full((1, F)),
            full((F, NCLS)), full((1, NCLS)),
        ],
        out_specs=[
            pl.BlockSpec((_ROWS_B, F), lambda i: (i, 0)),
            pl.BlockSpec((_ROWS_B, NCLS), lambda i: (i, 0)),
        ],
        out_shape=[
            jax.ShapeDtypeStruct((N, F), jnp.float32),
            jax.ShapeDtypeStruct((N, NCLS), jnp.float32),
        ],
    )(x, w1t, b1, w2t, b2, wl1t, bl1)


_SC_MESH = plsc.VectorSubcoreMesh(core_axis_name="c", subcore_axis_name="s")


def _prop_body(h_hbm, src_hbm, dst_hbm, z_hbm, out0, out1,
               sidx, didx, rows, acc, sem):
    c = lax.axis_index("c")
    s = lax.axis_index("s")
    w = s * 2 + c
    pltpu.sync_copy(z_hbm.at[pl.ds(s * RPT, RPT)], acc.at[pl.ds(s * RPT, RPT)])
    pltpu.sync_copy(src_hbm.at[w], sidx)
    pltpu.sync_copy(dst_hbm.at[w], didx)
    plsc.subcore_barrier()

    def chunk(j, carry):
        pltpu.async_copy(h_hbm.at[sidx.at[j]], rows, sem).wait()
        pltpu.sync_copy(rows, acc.at[didx.at[j]], add=True)
        return carry

    lax.fori_loop(0, CHUNKS, chunk, 0)
    plsc.subcore_barrier()

    @pl.when(c == 0)
    def _():
        pltpu.sync_copy(acc.at[pl.ds(s * RPT, RPT)], out0.at[pl.ds(s * RPT, RPT)])

    @pl.when(c == 1)
    def _():
        pltpu.sync_copy(acc.at[pl.ds(s * RPT, RPT)], out1.at[pl.ds(s * RPT, RPT)])


_prop = pl.kernel(
    _prop_body,
    out_type=[
        jax.ShapeDtypeStruct((NPAD, F), jnp.float32),
        jax.ShapeDtypeStruct((NPAD, F), jnp.float32),
    ],
    mesh=_SC_MESH,
    scratch_types=[
        pltpu.VMEM((CHUNKS, CH), jnp.int32),
        pltpu.VMEM((CHUNKS, CH), jnp.int32),
        pltpu.VMEM((CH, F), jnp.float32),
        pltpu.VMEM_SHARED((NPAD, F), jnp.float32),
        pltpu.SemaphoreType.DMA,
    ],
)


def _merge_body(p0_ref, p1_ref, o_ref):
    o_ref[...] = p0_ref[...] + p1_ref[...]


def _merge(p0, p1):
    return pl.pallas_call(
        _merge_body,
        grid=(N // _ROWS_B,),
        in_specs=[
            pl.BlockSpec((_ROWS_B, F), lambda i: (i, 0)),
            pl.BlockSpec((_ROWS_B, F), lambda i: (i, 0)),
        ],
        out_specs=pl.BlockSpec((_ROWS_B, F), lambda i: (i, 0)),
        out_shape=jax.ShapeDtypeStruct((N, F), jnp.float32),
    )(p0, p1)


def _post_body(h_ref, h1_ref, p0_ref, p1_ref, w3t_ref, b3_ref, w4t_ref,
               b4_ref, ix_ref, h3_ref, o_ref):
    h = h_ref[...]
    h1 = h1_ref[...]
    h2 = p0_ref[...] + p1_ref[...]
    t0, t1, t2 = _THETAS
    hf = jnp.concatenate(
        [t0[0] * h + t0[1] * h1 + t0[2] * h2,
         t1[1] * h1 + t1[2] * h2,
         t2[2] * h2], axis=1)
    h3 = jnp.maximum(
        jnp.dot(hf, w3t_ref[...], preferred_element_type=jnp.float32)
        + b3_ref[...], 0.0)
    h3_ref[...] = h3
    o_ref[...] = (jnp.dot(h3, w4t_ref[...], preferred_element_type=jnp.float32)
                  + b4_ref[...] + ix_ref[...])


def _post(h, h1, p0, p1, w3t, b3, w4t, b4, ix):
    full = lambda shape: pl.BlockSpec(shape, lambda i: (0, 0))
    row = lambda cols: pl.BlockSpec((_ROWS_B, cols), lambda i: (i, 0))
    return pl.pallas_call(
        _post_body,
        grid=(N // _ROWS_B,),
        in_specs=[
            row(F), row(F), row(F), row(F),
            full((3 * F, F)), full((1, F)),
            full((F, NCLS)), full((1, NCLS)),
            row(NCLS),
        ],
        out_specs=[row(F), row(NCLS)],
        out_shape=[
            jax.ShapeDtypeStruct((N, F), jnp.float32),
            jax.ShapeDtypeStruct((N, NCLS), jnp.float32),
        ],
    )(h, h1, p0, p1, w3t, b3, w4t, b4, ix)


def kernel(x, edge_index, W1, b1, W2, b2, W3, b3, W4, b4, Wl1, bl1):
    src = edge_index[0].astype(jnp.int32)
    dst = edge_index[1].astype(jnp.int32)
    pad = EPAD - E
    src3 = jnp.concatenate([src, jnp.zeros((pad,), jnp.int32)]).reshape(
        NWORK, CHUNKS, CH)
    dst3 = jnp.concatenate([dst, jnp.full((pad,), N, jnp.int32)]).reshape(
        NWORK, CHUNKS, CH)
    zeros = jnp.zeros((NPAD, F), jnp.float32)

    h, ix = _mlp(x, W1.T, b1.reshape(1, F), W2.T, b2.reshape(1, F),
                 Wl1.T, bl1.reshape(1, NCLS))
    p0, p1 = _prop(h, src3, dst3, zeros)
    h1 = _merge(p0, p1)
    q0, q1 = _prop(h1, src3, dst3, zeros)
    h3, out2 = _post(h, h1, q0, q1, W3.T, b3.reshape(1, F), W4.T,
                     b4.reshape(1, NCLS), ix)
    return (h3, out2)
